# baseline (speedup 1.0000x reference)
import functools

import jax
import jax.numpy as jnp
from jax.experimental import pallas as pl

NUM_EXPERTS = 8
TOP_K = 2
BM = 512


def _topk2_weights(logits):
    E = logits.shape[1]
    iota = jax.lax.broadcasted_iota(jnp.int32, logits.shape, 1)
    m1 = jnp.max(logits, axis=1, keepdims=True)
    eq1 = logits == m1
    idx1 = jnp.min(jnp.where(eq1, iota, E), axis=1, keepdims=True)
    first = iota == idx1
    masked = jnp.where(first, -jnp.inf, logits)
    m2 = jnp.max(masked, axis=1, keepdims=True)
    eq2 = masked == m2
    idx2 = jnp.min(jnp.where(eq2, iota, E), axis=1, keepdims=True)
    second = iota == idx2
    p = jnp.exp(m2 - m1)
    denom = 1.0 + p
    w = (first.astype(jnp.float32) + second.astype(jnp.float32) * p) / denom
    return w


def _moe_body(x_ref, xb_ref, rw_ref, rb_ref, ew_ref, eb_ref, out_ref):
    e = pl.program_id(1)
    logits = jax.lax.dot_general(
        x_ref[...], rw_ref[...],
        (((1,), (1,)), ((), ())),
        preferred_element_type=jnp.float32,
    ) + rb_ref[...]
    w = _topk2_weights(logits)
    onehot = (jax.lax.broadcasted_iota(jnp.int32, w.shape, 1) == e)
    w_e = jnp.sum(jnp.where(onehot, w, 0.0), axis=1, keepdims=True)

    @pl.when(e == 0)
    def _():
        out_ref[...] = jnp.zeros_like(out_ref)

    y = jax.lax.dot_general(
        xb_ref[...], ew_ref[0],
        (((1,), (1,)), ((), ())),
        preferred_element_type=jnp.float32,
    ) + eb_ref[0]
    out_ref[...] += y * w_e


@functools.partial(jax.jit, static_argnames=("interpret",))
def kernel(x, router_W, router_b, expert_weight, expert_bias, interpret=False):
    B, D = x.shape
    E, O, _ = expert_weight.shape
    rb2 = router_b.reshape(1, E)
    eb3 = expert_bias.reshape(E, 1, O)
    xb = x.astype(jnp.bfloat16)
    ewb = expert_weight.astype(jnp.bfloat16)
    grid = (B // BM, E)
    out = pl.pallas_call(
        _moe_body,
        grid=grid,
        in_specs=[
            pl.BlockSpec((BM, D), lambda t, e: (t, 0)),
            pl.BlockSpec((BM, D), lambda t, e: (t, 0)),
            pl.BlockSpec((E, D), lambda t, e: (0, 0)),
            pl.BlockSpec((1, E), lambda t, e: (0, 0)),
            pl.BlockSpec((1, O, D), lambda t, e: (e, 0, 0)),
            pl.BlockSpec((1, 1, O), lambda t, e: (e, 0, 0)),
        ],
        out_specs=pl.BlockSpec((BM, O), lambda t, e: (t, 0)),
        out_shape=jax.ShapeDtypeStruct((B, O), jnp.float32),
        interpret=interpret,
    )(x, xb, router_W, rb2, ewb, eb3)
    return out


# speedup vs baseline: 1.2470x; 1.2470x over previous
import functools

import jax
import jax.numpy as jnp
from jax.experimental import pallas as pl
from jax.experimental.pallas import tpu as pltpu

NUM_EXPERTS = 8
TOP_K = 2
BM = 512


def _topk2_weights(logits):
    E = logits.shape[1]
    iota = jax.lax.broadcasted_iota(jnp.int32, logits.shape, 1)
    m1 = jnp.max(logits, axis=1, keepdims=True)
    eq1 = logits == m1
    idx1 = jnp.min(jnp.where(eq1, iota, E), axis=1, keepdims=True)
    first = iota == idx1
    masked = jnp.where(first, -jnp.inf, logits)
    m2 = jnp.max(masked, axis=1, keepdims=True)
    eq2 = masked == m2
    idx2 = jnp.min(jnp.where(eq2, iota, E), axis=1, keepdims=True)
    second = iota == idx2
    p = jnp.exp(m2 - m1)
    denom = 1.0 + p
    w = (first.astype(jnp.float32) + second.astype(jnp.float32) * p) / denom
    return w


def _moe_body(x_ref, rw_ref, rb_ref, ew_ref, eb_ref, out_ref, w_scr):
    e = pl.program_id(1)

    @pl.when(e == 0)
    def _():
        logits = jax.lax.dot_general(
            x_ref[...], rw_ref[...],
            (((1,), (1,)), ((), ())),
            preferred_element_type=jnp.float32,
        ) + rb_ref[...]
        w_scr[...] = _topk2_weights(logits)
        out_ref[...] = jnp.zeros_like(out_ref)

    w_all = w_scr[...]
    onehot = jax.lax.broadcasted_iota(jnp.int32, w_all.shape, 1) == e
    w_e = jnp.sum(jnp.where(onehot, w_all, 0.0), axis=1, keepdims=True)
    y = jax.lax.dot_general(
        x_ref[...], ew_ref[0],
        (((1,), (1,)), ((), ())),
        preferred_element_type=jnp.float32,
    ) + eb_ref[0]
    out_ref[...] += y * w_e


@functools.partial(jax.jit, static_argnames=("interpret",))
def kernel(x, router_W, router_b, expert_weight, expert_bias, interpret=False):
    B, D = x.shape
    E, O, _ = expert_weight.shape
    rb2 = router_b.reshape(1, E)
    eb3 = expert_bias.reshape(E, 1, O)
    grid = (B // BM, E)
    out = pl.pallas_call(
        _moe_body,
        grid=grid,
        in_specs=[
            pl.BlockSpec((BM, D), lambda t, e: (t, 0)),
            pl.BlockSpec((E, D), lambda t, e: (0, 0)),
            pl.BlockSpec((1, E), lambda t, e: (0, 0)),
            pl.BlockSpec((1, O, D), lambda t, e: (e, 0, 0)),
            pl.BlockSpec((1, 1, O), lambda t, e: (e, 0, 0)),
        ],
        out_specs=pl.BlockSpec((BM, O), lambda t, e: (t, 0)),
        out_shape=jax.ShapeDtypeStruct((B, O), jnp.float32),
        scratch_shapes=[pltpu.VMEM((BM, NUM_EXPERTS), jnp.float32)],
        interpret=interpret,
    )(x, router_W, rb2, expert_weight, eb3)
    return out
